# baseline (speedup 1.0000x reference)
import functools

import jax
import jax.numpy as jnp
from jax import lax
from jax.experimental import pallas as pl
from jax.experimental.pallas import tpu as pltpu
from jax.experimental.pallas import tpu_sc as plsc

EMBED = 32
B = 16384
H1 = 128
H2 = 64
NC = 2
NS = 16
NW = NC * NS
BPW = B // NW
CH = 128
NCH = BPW // CH


def _sc_gather(uid2, iid2, Ug, Ig, Um, Im, ub, ib):
    mesh = plsc.VectorSubcoreMesh(core_axis_name="c", subcore_axis_name="s")

    @functools.partial(
        pl.kernel,
        mesh=mesh,
        compiler_params=pltpu.CompilerParams(use_tc_tiling_on_sc=False),
        out_type=[
            jax.ShapeDtypeStruct((B, EMBED), jnp.float32),
            jax.ShapeDtypeStruct((B, EMBED), jnp.float32),
            jax.ShapeDtypeStruct((B, EMBED), jnp.float32),
            jax.ShapeDtypeStruct((B, EMBED), jnp.float32),
            jax.ShapeDtypeStruct((B,), jnp.float32),
            jax.ShapeDtypeStruct((B,), jnp.float32),
        ],
        scratch_types=[
            pltpu.VMEM((NCH, CH), jnp.int32),
            pltpu.VMEM((NCH, CH), jnp.int32),
            pltpu.VMEM((BPW, EMBED), jnp.float32),
            pltpu.VMEM((BPW, EMBED), jnp.float32),
            pltpu.VMEM((BPW, EMBED), jnp.float32),
            pltpu.VMEM((BPW, EMBED), jnp.float32),
            pltpu.VMEM((BPW,), jnp.float32),
            pltpu.VMEM((BPW,), jnp.float32),
            pltpu.SemaphoreType.DMA,
            pltpu.SemaphoreType.DMA,
        ],
    )
    def k(uid_h, iid_h, ug_h, ig_h, um_h, im_h, ub_h, ib_h,
          oug, oig, oum, oim, obu, obi,
          uidx, iidx, vug, vig, vum, vim, vbu, vbi, gsem, wsem):
        wid = lax.axis_index("s") * NC + lax.axis_index("c")
        base = wid * BPW
        pltpu.sync_copy(uid_h.at[pl.ds(wid * NCH, NCH)], uidx)
        pltpu.sync_copy(iid_h.at[pl.ds(wid * NCH, NCH)], iidx)
        gathers = []
        for j in range(NCH):
            sl = pl.ds(j * CH, CH)
            gathers.append(pltpu.async_copy(ug_h.at[uidx.at[j]], vug.at[sl], gsem))
            gathers.append(pltpu.async_copy(ig_h.at[iidx.at[j]], vig.at[sl], gsem))
            gathers.append(pltpu.async_copy(um_h.at[uidx.at[j]], vum.at[sl], gsem))
            gathers.append(pltpu.async_copy(im_h.at[iidx.at[j]], vim.at[sl], gsem))
            gathers.append(pltpu.async_copy(ub_h.at[uidx.at[j]], vbu.at[sl], gsem))
            gathers.append(pltpu.async_copy(ib_h.at[iidx.at[j]], vbi.at[sl], gsem))
        for g in gathers:
            g.wait()
        writes = [
            pltpu.async_copy(vug, oug.at[pl.ds(base, BPW)], wsem),
            pltpu.async_copy(vig, oig.at[pl.ds(base, BPW)], wsem),
            pltpu.async_copy(vum, oum.at[pl.ds(base, BPW)], wsem),
            pltpu.async_copy(vim, oim.at[pl.ds(base, BPW)], wsem),
            pltpu.async_copy(vbu, obu.at[pl.ds(base, BPW)], wsem),
            pltpu.async_copy(vbi, obi.at[pl.ds(base, BPW)], wsem),
        ]
        for w in writes:
            w.wait()

    return k(uid2, iid2, Ug, Ig, Um, Im, ub, ib)


def _tc_mlp(ug, ig, um, im, bu2, bi2, w1u, w1i, b1r, W2, b2r, wog, woh, bo):
    BLK = BPW
    G = B // BLK

    def body(ug_r, ig_r, um_r, im_r, bu_r, bi_r, w1u_r, w1i_r, b1_r,
             w2_r, b2_r, wog_r, woh_r, bo_r, out_r):
        g = ug_r[...] * ig_r[...]
        x1 = jnp.dot(um_r[...], w1u_r[...], preferred_element_type=jnp.float32)
        x1 = x1 + jnp.dot(im_r[...], w1i_r[...], preferred_element_type=jnp.float32)
        h1 = jnp.maximum(x1 + b1_r[...], 0.0)
        x2 = jnp.dot(h1, w2_r[...], preferred_element_type=jnp.float32)
        h2 = jnp.maximum(x2 + b2_r[...], 0.0)
        p = jnp.sum(g * wog_r[...], axis=1) + jnp.sum(h2 * woh_r[...], axis=1)
        out_r[...] = (p + bo_r[0]).reshape(1, 1, BLK) + bu_r[...] + bi_r[...]

    out = pl.pallas_call(
        body,
        grid=(G,),
        in_specs=[
            pl.BlockSpec((BLK, EMBED), lambda i: (i, 0)),
            pl.BlockSpec((BLK, EMBED), lambda i: (i, 0)),
            pl.BlockSpec((BLK, EMBED), lambda i: (i, 0)),
            pl.BlockSpec((BLK, EMBED), lambda i: (i, 0)),
            pl.BlockSpec((1, 1, BLK), lambda i: (i, 0, 0)),
            pl.BlockSpec((1, 1, BLK), lambda i: (i, 0, 0)),
            pl.BlockSpec((EMBED, H1), lambda i: (0, 0)),
            pl.BlockSpec((EMBED, H1), lambda i: (0, 0)),
            pl.BlockSpec((1, H1), lambda i: (0, 0)),
            pl.BlockSpec((H1, H2), lambda i: (0, 0)),
            pl.BlockSpec((1, H2), lambda i: (0, 0)),
            pl.BlockSpec((1, EMBED), lambda i: (0, 0)),
            pl.BlockSpec((1, H2), lambda i: (0, 0)),
            pl.BlockSpec(memory_space=pltpu.SMEM),
        ],
        out_specs=pl.BlockSpec((1, 1, BLK), lambda i: (i, 0, 0)),
        out_shape=jax.ShapeDtypeStruct((G, 1, BLK), jnp.float32),
    )(ug, ig, um, im, bu2, bi2, w1u, w1i, b1r, W2, b2r, wog, woh, bo)
    return out.reshape(B)


def kernel(user_ids, item_ids, Ug, Ig, Um, Im, Ub, Ib, W1, b1, W2, b2, Wo, bo):
    uid2 = user_ids.astype(jnp.int32).reshape(B // CH, CH)
    iid2 = item_ids.astype(jnp.int32).reshape(B // CH, CH)
    ug, ig, um, im, bu, bi = _sc_gather(
        uid2, iid2, Ug, Ig, Um, Im, Ub.reshape(-1), Ib.reshape(-1))
    return _tc_mlp(
        ug, ig, um, im,
        bu.reshape(B // BPW, 1, BPW), bi.reshape(B // BPW, 1, BPW),
        W1[:EMBED], W1[EMBED:], b1.reshape(1, H1),
        W2, b2.reshape(1, H2),
        Wo[:EMBED].reshape(1, EMBED), Wo[EMBED:].reshape(1, H2), bo)
